# Initial kernel scaffold; baseline (speedup 1.0000x reference)
import jax
import jax.numpy as jnp
from jax.experimental import pallas as pl


def kernel(token_ids, weight):
    raise NotImplementedError("write your pallas kernel here")


# speedup vs baseline: 1.1093x; 1.1093x over previous
import functools

import jax
import jax.numpy as jnp
from jax import lax
from jax.experimental import pallas as pl
from jax.experimental.pallas import tpu as pltpu
from jax.experimental.pallas import tpu_sc as plsc

NUM_EMB = 1000000
DIM = 32
B_TOK = 16384
SEQ = 50
B = B_TOK * SEQ

NC = 2
NS = 16
NW = NC * NS
B_PER_W = B // NW
CHUNK = 128
N_CHUNKS = B_PER_W // CHUNK
K = 8


def _emb_body(idx_hbm, table_hbm, out_hbm, idx_v, rows_v, gsem):
    wid = lax.axis_index("s") * NC + lax.axis_index("c")
    pltpu.sync_copy(idx_hbm.at[wid], idx_v)
    base = wid * B_PER_W

    def outer(g, carry):
        off = g * K
        cps = []
        for j in range(K):
            cps.append(
                pltpu.async_copy(table_hbm.at[idx_v.at[off + j]], rows_v.at[j], gsem)
            )
        for j in range(K):
            cps[j].wait()
            pltpu.sync_copy(
                rows_v.at[j], out_hbm.at[pl.ds(base + (off + j) * CHUNK, CHUNK)]
            )
        return carry

    lax.fori_loop(0, N_CHUNKS // K, outer, 0)


@jax.jit
def _emb(idx3d, weight):
    mesh = plsc.VectorSubcoreMesh(core_axis_name="c", subcore_axis_name="s")
    run = pl.kernel(
        _emb_body,
        out_type=jax.ShapeDtypeStruct((B, DIM), jnp.float32),
        mesh=mesh,
        scratch_types=[
            pltpu.VMEM((N_CHUNKS, CHUNK), jnp.int32),
            pltpu.VMEM((K, CHUNK, DIM), jnp.float32),
            pltpu.SemaphoreType.DMA,
        ],
        compiler_params=pltpu.CompilerParams(use_tc_tiling_on_sc=False),
    )
    return run(idx3d, weight)


def kernel(token_ids, weight):
    idx = token_ids.astype(jnp.int32).reshape(NW, N_CHUNKS, CHUNK)
    out = _emb(idx, weight)
    return out.reshape(B_TOK, SEQ, DIM)


# speedup vs baseline: 1.1144x; 1.0045x over previous
import functools

import jax
import jax.numpy as jnp
from jax import lax
from jax.experimental import pallas as pl
from jax.experimental.pallas import tpu as pltpu
from jax.experimental.pallas import tpu_sc as plsc

NUM_EMB = 1000000
DIM = 32
B_TOK = 16384
SEQ = 50
B = B_TOK * SEQ

NC = 2
NS = 16
NW = NC * NS
B_PER_W = B // NW
CHUNK = 128
N_CHUNKS = B_PER_W // CHUNK
K = 8
NGRP = N_CHUNKS // K


def _emb_body(idx_hbm, table_hbm, out_hbm, idx_v, rows_v, gsem, ssem):
    wid = lax.axis_index("s") * NC + lax.axis_index("c")
    pltpu.sync_copy(idx_hbm.at[wid], idx_v)
    base = wid * B_PER_W

    def fire_gathers(g, buf):
        for j in range(K):
            pltpu.async_copy(
                table_hbm.at[idx_v.at[g * K + j]], rows_v.at[buf, j], gsem
            )

    def drain_gathers(g, buf):
        for j in range(K):
            pltpu.make_async_copy(
                table_hbm.at[idx_v.at[g * K + j]], rows_v.at[buf, j], gsem
            ).wait()

    def fire_stores(g, buf):
        for j in range(K):
            pltpu.async_copy(
                rows_v.at[buf, j],
                out_hbm.at[pl.ds(base + (g * K + j) * CHUNK, CHUNK)],
                ssem,
            )

    def drain_stores(g, buf):
        for j in range(K):
            pltpu.make_async_copy(
                rows_v.at[buf, j],
                out_hbm.at[pl.ds(base + (g * K + j) * CHUNK, CHUNK)],
                ssem,
            ).wait()

    fire_gathers(0, 0)
    fire_gathers(1, 1)
    drain_gathers(0, 0)
    fire_stores(0, 0)

    def body(g, carry):
        b_cur = lax.rem(g, 2)
        b_nxt = lax.rem(g + 1, 2)
        drain_stores(g - 1, b_nxt)
        fire_gathers(g + 1, b_nxt)
        drain_gathers(g, b_cur)
        fire_stores(g, b_cur)
        return carry

    lax.fori_loop(1, NGRP - 1, body, 0)

    g_last = NGRP - 1
    b_last = g_last % 2
    drain_stores(g_last - 1, 1 - b_last)
    drain_gathers(g_last, b_last)
    fire_stores(g_last, b_last)
    drain_stores(g_last, b_last)


@jax.jit
def _emb(idx3d, weight):
    mesh = plsc.VectorSubcoreMesh(core_axis_name="c", subcore_axis_name="s")
    run = pl.kernel(
        _emb_body,
        out_type=jax.ShapeDtypeStruct((B, DIM), jnp.float32),
        mesh=mesh,
        scratch_types=[
            pltpu.VMEM((N_CHUNKS, CHUNK), jnp.int32),
            pltpu.VMEM((2, K, CHUNK, DIM), jnp.float32),
            pltpu.SemaphoreType.DMA,
            pltpu.SemaphoreType.DMA,
        ],
        compiler_params=pltpu.CompilerParams(use_tc_tiling_on_sc=False),
    )
    return run(idx3d, weight)


def kernel(token_ids, weight):
    idx = token_ids.astype(jnp.int32).reshape(NW, N_CHUNKS, CHUNK)
    out = _emb(idx, weight)
    return out.reshape(B_TOK, SEQ, DIM)


# speedup vs baseline: 1.8016x; 1.6167x over previous
import functools

import jax
import jax.numpy as jnp
from jax import lax
from jax.experimental import pallas as pl
from jax.experimental.pallas import tpu as pltpu
from jax.experimental.pallas import tpu_sc as plsc

NUM_EMB = 1000000
DIM = 32
B_TOK = 16384
SEQ = 50

NC = 2
NS = 16
NW = NC * NS
TOK_PER_W = B_TOK // NW
T = 8
NGRP = TOK_PER_W // T


def _emb_body(idx_hbm, table_hbm, out_hbm, idx_v, rows_v, gsem, ssem):
    wid = lax.axis_index("s") * NC + lax.axis_index("c")
    tok0 = wid * TOK_PER_W
    pltpu.sync_copy(idx_hbm.at[pl.ds(tok0, TOK_PER_W)], idx_v)

    def fire_gather(g, buf):
        for j in range(T):
            pltpu.async_copy(
                table_hbm.at[idx_v.at[g * T + j]], rows_v.at[buf, j], gsem
            )

    def drain_gather(g, buf):
        for j in range(T):
            pltpu.make_async_copy(
                table_hbm.at[idx_v.at[g * T + j]], rows_v.at[buf, j], gsem
            ).wait()

    def fire_store(g, buf):
        pltpu.async_copy(
            rows_v.at[buf], out_hbm.at[pl.ds(tok0 + g * T, T)], ssem
        )

    def drain_store(g, buf):
        pltpu.make_async_copy(
            rows_v.at[buf], out_hbm.at[pl.ds(tok0 + g * T, T)], ssem
        ).wait()

    fire_gather(0, 0)
    fire_gather(1, 1)
    drain_gather(0, 0)
    fire_store(0, 0)

    def body(g, carry):
        b_cur = lax.rem(g, 2)
        b_nxt = lax.rem(g + 1, 2)
        drain_store(g - 1, b_nxt)
        fire_gather(g + 1, b_nxt)
        drain_gather(g, b_cur)
        fire_store(g, b_cur)
        return carry

    lax.fori_loop(1, NGRP - 1, body, 0)

    g_last = NGRP - 1
    b_last = g_last % 2
    drain_store(g_last - 1, 1 - b_last)
    drain_gather(g_last, b_last)
    fire_store(g_last, b_last)
    drain_store(g_last, b_last)


@jax.jit
def _emb(idx, weight):
    mesh = plsc.VectorSubcoreMesh(core_axis_name="c", subcore_axis_name="s")
    run = pl.kernel(
        _emb_body,
        out_type=jax.ShapeDtypeStruct((B_TOK, SEQ, DIM), jnp.float32),
        mesh=mesh,
        scratch_types=[
            pltpu.VMEM((TOK_PER_W, SEQ), jnp.int32),
            pltpu.VMEM((2, T, SEQ, DIM), jnp.float32),
            pltpu.SemaphoreType.DMA,
            pltpu.SemaphoreType.DMA,
        ],
        compiler_params=pltpu.CompilerParams(use_tc_tiling_on_sc=False),
    )
    return run(idx, weight)


def kernel(token_ids, weight):
    return _emb(token_ids.astype(jnp.int32), weight)
